# baseline (speedup 1.0000x reference)
import functools

import jax
import jax.numpy as jnp
from jax import lax
from jax.experimental import pallas as pl
from jax.experimental.pallas import tpu as pltpu
from jax.experimental.pallas import tpu_sc as plsc

B = 128
V = 100000
C = 0.1
VB = 50000
K = V // VB
NWORK = B // 16


def _sc_gather_body(target_hbm, predt_hbm, out_hbm, tgt_v, rows_v, diag_v, sem):
    wid = lax.axis_index("s") * 2 + lax.axis_index("c")

    @pl.when(wid < NWORK)
    def _():
        base = wid * 16
        pltpu.sync_copy(target_hbm.at[pl.ds(base, 16)], tgt_v)
        pltpu.async_copy(predt_hbm.at[tgt_v], rows_v, sem).wait()
        ii = lax.iota(jnp.int32, 16)
        d = jnp.full((16,), -C, jnp.float32)
        for l in range(16):
            vec = rows_v[l, pl.ds(base, 16)]
            d = jnp.where(ii == l, vec - C, d)
        diag_v[...] = d
        pltpu.sync_copy(diag_v, out_hbm.at[pl.ds(base, 16)])


_sc_gather = functools.partial(
    pl.kernel,
    mesh=plsc.VectorSubcoreMesh(core_axis_name="c", subcore_axis_name="s"),
    out_type=jax.ShapeDtypeStruct((B,), jnp.float32),
    scratch_types=[
        pltpu.VMEM((16,), jnp.int32),
        pltpu.VMEM((16, B), jnp.float32),
        pltpu.VMEM((16,), jnp.float32),
        pltpu.SemaphoreType.DMA,
    ],
    compiler_params=pltpu.CompilerParams(use_tc_tiling_on_sc=True),
)(_sc_gather_body)


def _tc_body(corrc_ref, pred_ref, out_ref, acc_ref):
    k = pl.program_id(0)
    s = jnp.sum(jnp.maximum(pred_ref[...] - corrc_ref[...], 0.0))

    @pl.when(k == 0)
    def _():
        acc_ref[0] = s

    @pl.when(k > 0)
    def _():
        acc_ref[0] += s

    @pl.when(k == K - 1)
    def _():
        out_ref[0] = (acc_ref[0] - B * C) / B


def kernel(target, prediction):
    target = target.astype(jnp.int32)
    predt = prediction.T
    corrc = _sc_gather(target, predt)
    out = pl.pallas_call(
        _tc_body,
        grid=(K,),
        in_specs=[
            pl.BlockSpec((1, B), lambda k: (0, 0)),
            pl.BlockSpec((VB, B), lambda k: (k, 0)),
        ],
        out_specs=pl.BlockSpec(memory_space=pltpu.SMEM),
        out_shape=jax.ShapeDtypeStruct((1,), jnp.float32),
        scratch_shapes=[pltpu.SMEM((1,), jnp.float32)],
    )(corrc.reshape(1, B), predt)
    return out
